# baseline (speedup 1.0000x reference)
import jax
import jax.numpy as jnp
from jax import lax
from jax.experimental import pallas as pl
from jax.experimental.pallas import tpu as pltpu
from jax.experimental.pallas import tpu_sc as plsc

N = 128
L = 200
PATCH = 16
IMG = 384
EMB_DIM = 768
ROWS = 24
LANES = 16
VECS = EMB_DIM // LANES

NUM_CORES = 2
NUM_SUBCORES = 16
NUM_WORKERS = NUM_CORES * NUM_SUBCORES


def _tanh16(x):
  return 1.0 - 2.0 / (jnp.exp(2.0 * x) + 1.0)


SPW = 2


def _sc_body(tok_hbm, table_hbm, out_hbm,
             tok_v, rows0, rows1, rows8, band_a0, band_a1, band_b,
             gs0, gs1, gs8, ws0, ws1, rsem):
  wid = lax.axis_index("s") * NUM_CORES + lax.axis_index("c")
  rows_bufs = (rows0, rows1)
  gsems = (gs0, gs1)
  band_bufs = (band_a0, band_a1)
  wsems = (ws0, ws1)

  tok_off = pl.multiple_of(wid * (SPW * L), 8)
  pltpu.sync_copy(tok_hbm.at[pl.ds(tok_off, SPW * L)], tok_v)

  def idx(i, start, count):
    return tok_v.at[pl.ds(pl.multiple_of(i * L + start, 8), count)]

  def gather(i, r, buf, sem):
    if r < 8:
      return pltpu.make_async_copy(table_hbm.at[idx(i, r * ROWS, ROWS)],
                                   buf, sem)
    return pltpu.make_async_copy(table_hbm.at[idx(i, 192, 8)], buf, sem)

  def band_write(n, r, buf, sem):
    return pltpu.make_async_copy(
        buf, out_hbm.at[n, :, pl.ds(pl.multiple_of(r * PATCH, 16), PATCH), :],
        sem)

  def build_cols(rows, band, lo, hi):
    def do_col(c, _):
      cbase = pl.multiple_of(c * PATCH, 16)

      @plsc.parallel_loop(0, VECS, unroll=4)
      def _(j):
        ch = j // PATCH
        y = j - ch * PATCH
        band[ch, y, pl.ds(cbase, LANES)] = _tanh16(
            rows[c, pl.ds(pl.multiple_of(j * LANES, 16), LANES)])

      return 0
    lax.fori_loop(lo, hi, do_col, 0)

  def do_sample(i, _):
    n = wid * SPW + i
    gather(i, 0, rows0, gs0).start()

    for r in range(8):
      if r < 7:
        gather(i, r + 1, rows_bufs[(r + 1) % 2], gsems[(r + 1) % 2]).start()
      else:
        gather(i, 8, rows8, gs8).start()
      gather(i, r, rows_bufs[r % 2], gsems[r % 2]).wait()

      if r >= 2:
        band_write(n, r, band_bufs[r % 2], wsems[r % 2]).wait()
      else:
        @pl.when(i > 0)
        def _():
          band_write(n, r, band_bufs[r % 2], wsems[r % 2]).wait()

      build_cols(rows_bufs[r % 2], band_bufs[r % 2], 0, ROWS)
      band_write(n, r, band_bufs[r % 2], wsems[r % 2]).start()

    gather(i, 8, rows8, gs8).wait()

    @pl.when(i > 0)
    def _():
      for _k in range(ROWS - 9):
        band_write(n, 9, band_b, rsem).wait()

    @plsc.parallel_loop(0, VECS, unroll=4)
    def _(j):
      ch = j // PATCH
      y = j - ch * PATCH
      band_b[ch, y, pl.ds(0, LANES)] = _tanh16(
          rows8[7, pl.ds(pl.multiple_of(j * LANES, 16), LANES)])

    def fan_col(c, _):
      cbase = pl.multiple_of(c * PATCH, 16)

      @plsc.parallel_loop(0, VECS, unroll=4)
      def _(j):
        ch = j // PATCH
        y = j - ch * PATCH
        band_b[ch, y, pl.ds(cbase, LANES)] = band_b[ch, y, pl.ds(0, LANES)]

      return 0
    lax.fori_loop(1, ROWS, fan_col, 0)

    def write_rep(r, _):
      band_write(n, r, band_b, rsem).start()
      return 0
    lax.fori_loop(9, ROWS, write_rep, 0)

    band_write(n, 8, band_a0, ws0).wait()
    build_cols(rows8, band_a0, 0, 7)

    def fill_199(c, _):
      cbase = pl.multiple_of(c * PATCH, 16)

      @plsc.parallel_loop(0, VECS, unroll=4)
      def _(j):
        ch = j // PATCH
        y = j - ch * PATCH
        band_a0[ch, y, pl.ds(cbase, LANES)] = band_b[ch, y, pl.ds(0, LANES)]

      return 0
    lax.fori_loop(7, ROWS, fill_199, 0)
    band_write(n, 8, band_a0, ws0).start()
    return 0

  lax.fori_loop(0, SPW, do_sample, 0)

  last = wid * SPW + (SPW - 1)
  band_write(last, 8, band_a0, ws0).wait()
  band_write(last, 7, band_a1, ws1).wait()
  for _k in range(ROWS - 9):
    band_write(last, 9, band_b, rsem).wait()


@jax.jit
def kernel(sentence_batch, emb_table):
  mesh = plsc.VectorSubcoreMesh(core_axis_name="c", subcore_axis_name="s",
                                num_cores=NUM_CORES,
                                num_subcores=NUM_SUBCORES)
  run = pl.kernel(
      _sc_body,
      out_type=jax.ShapeDtypeStruct((N // 2, 3, IMG, IMG), jnp.float32),
      mesh=mesh,
      scratch_types=[
          pltpu.VMEM((SPW * L,), jnp.int32),
          pltpu.VMEM((ROWS, EMB_DIM), jnp.float32),
          pltpu.VMEM((ROWS, EMB_DIM), jnp.float32),
          pltpu.VMEM((8, EMB_DIM), jnp.float32),
          pltpu.VMEM((3, PATCH, IMG), jnp.float32),
          pltpu.VMEM((3, PATCH, IMG), jnp.float32),
          pltpu.VMEM((3, PATCH, IMG), jnp.float32),
          pltpu.SemaphoreType.DMA,
          pltpu.SemaphoreType.DMA,
          pltpu.SemaphoreType.DMA,
          pltpu.SemaphoreType.DMA,
          pltpu.SemaphoreType.DMA,
          pltpu.SemaphoreType.DMA,
      ],
  )
  toks = sentence_batch.astype(jnp.int32).reshape(N * L)
  lo = run(toks[: N * L // 2], emb_table)
  hi = run(toks[N * L // 2:], emb_table)
  return jnp.concatenate([lo, hi], axis=0)


# speedup vs baseline: 1.9293x; 1.9293x over previous
import jax
import jax.numpy as jnp
from jax import lax
from jax.experimental import pallas as pl
from jax.experimental.pallas import tpu as pltpu
from jax.experimental.pallas import tpu_sc as plsc

N = 128
L = 200
PATCH = 16
IMG = 384
EMB_DIM = 768
ROWS = 24
LANES = 16
VECS = EMB_DIM // LANES

NUM_CORES = 2
NUM_SUBCORES = 16
NUM_WORKERS = NUM_CORES * NUM_SUBCORES


def _tanh16(x):
  return 1.0 - 2.0 / (jnp.exp(2.0 * x) + 1.0)


SPW = N // NUM_WORKERS


def _sc_body(tok_hbm, table_hbm, out_hbm,
             tok_v, rows0, rows1, rows8, band_a0, band_a1, band_b,
             gs0, gs1, gs8, ws0, ws1, rsem):
  wid = lax.axis_index("s") * NUM_CORES + lax.axis_index("c")
  rows_bufs = (rows0, rows1)
  gsems = (gs0, gs1)
  band_bufs = (band_a0, band_a1)
  wsems = (ws0, ws1)

  tok_off = pl.multiple_of(wid * (SPW * L), 8)
  pltpu.sync_copy(tok_hbm.at[pl.ds(tok_off, SPW * L)], tok_v)

  def idx(i, start, count):
    return tok_v.at[pl.ds(pl.multiple_of(i * L + start, 8), count)]

  def gather(i, r, buf, sem):
    if r < 8:
      return pltpu.make_async_copy(table_hbm.at[idx(i, r * ROWS, ROWS)],
                                   buf, sem)
    return pltpu.make_async_copy(table_hbm.at[idx(i, 192, 8)], buf, sem)

  def band_write(n, r, buf, sem):
    return pltpu.make_async_copy(
        buf, out_hbm.at[n, :, pl.ds(pl.multiple_of(r * PATCH, 16), PATCH), :],
        sem)

  def build_cols(rows, band, lo, hi):
    def do_col(c, _):
      cbase = pl.multiple_of(c * PATCH, 16)

      @plsc.parallel_loop(0, VECS, unroll=4)
      def _(j):
        ch = j // PATCH
        y = j - ch * PATCH
        band[ch, y, pl.ds(cbase, LANES)] = _tanh16(
            rows[c, pl.ds(pl.multiple_of(j * LANES, 16), LANES)])

      return 0
    lax.fori_loop(lo, hi, do_col, 0)

  def do_sample(i, _):
    n = wid * SPW + i

    for r in range(8):
      if r < 7:
        gather(i, r + 1, rows_bufs[(r + 1) % 2], gsems[(r + 1) % 2]).start()
      else:
        gather(i, 8, rows8, gs8).start()
      gather(i, r, rows_bufs[r % 2], gsems[r % 2]).wait()

      if r >= 2:
        band_write(n, r, band_bufs[r % 2], wsems[r % 2]).wait()
      else:
        @pl.when(i > 0)
        def _():
          band_write(n, r, band_bufs[r % 2], wsems[r % 2]).wait()

      build_cols(rows_bufs[r % 2], band_bufs[r % 2], 0, ROWS)
      band_write(n, r, band_bufs[r % 2], wsems[r % 2]).start()

    gather(i, 8, rows8, gs8).wait()

    @pl.when(i + 1 < SPW)
    def _():
      gather(i + 1, 0, rows0, gs0).start()

    @pl.when(i > 0)
    def _():
      for _k in range(ROWS - 9):
        band_write(n, 9, band_b, rsem).wait()

    @plsc.parallel_loop(0, VECS, unroll=4)
    def _(j):
      ch = j // PATCH
      y = j - ch * PATCH
      band_b[ch, y, pl.ds(0, LANES)] = _tanh16(
          rows8[7, pl.ds(pl.multiple_of(j * LANES, 16), LANES)])

    def fan_col(c, _):
      cbase = pl.multiple_of(c * PATCH, 16)

      @plsc.parallel_loop(0, VECS, unroll=4)
      def _(j):
        ch = j // PATCH
        y = j - ch * PATCH
        band_b[ch, y, pl.ds(cbase, LANES)] = band_b[ch, y, pl.ds(0, LANES)]

      return 0
    lax.fori_loop(1, ROWS, fan_col, 0)

    def write_rep(r, _):
      band_write(n, r, band_b, rsem).start()
      return 0
    lax.fori_loop(9, ROWS, write_rep, 0)

    band_write(n, 8, band_a0, ws0).wait()
    build_cols(rows8, band_a0, 0, 7)

    def fill_199(c, _):
      cbase = pl.multiple_of(c * PATCH, 16)

      @plsc.parallel_loop(0, VECS, unroll=4)
      def _(j):
        ch = j // PATCH
        y = j - ch * PATCH
        band_a0[ch, y, pl.ds(cbase, LANES)] = band_b[ch, y, pl.ds(0, LANES)]

      return 0
    lax.fori_loop(7, ROWS, fill_199, 0)
    band_write(n, 8, band_a0, ws0).start()
    return 0

  gather(0, 0, rows0, gs0).start()
  lax.fori_loop(0, SPW, do_sample, 0)

  last = wid * SPW + (SPW - 1)
  band_write(last, 8, band_a0, ws0).wait()
  band_write(last, 7, band_a1, ws1).wait()
  for _k in range(ROWS - 9):
    band_write(last, 9, band_b, rsem).wait()


@jax.jit
def kernel(sentence_batch, emb_table):
  mesh = plsc.VectorSubcoreMesh(core_axis_name="c", subcore_axis_name="s",
                                num_cores=NUM_CORES,
                                num_subcores=NUM_SUBCORES)
  run = pl.kernel(
      _sc_body,
      out_type=jax.ShapeDtypeStruct((N, 3, IMG, IMG), jnp.float32),
      mesh=mesh,
      scratch_types=[
          pltpu.VMEM((SPW * L,), jnp.int32),
          pltpu.VMEM((ROWS, EMB_DIM), jnp.float32),
          pltpu.VMEM((ROWS, EMB_DIM), jnp.float32),
          pltpu.VMEM((8, EMB_DIM), jnp.float32),
          pltpu.VMEM((3, PATCH, IMG), jnp.float32),
          pltpu.VMEM((3, PATCH, IMG), jnp.float32),
          pltpu.VMEM((3, PATCH, IMG), jnp.float32),
          pltpu.SemaphoreType.DMA,
          pltpu.SemaphoreType.DMA,
          pltpu.SemaphoreType.DMA,
          pltpu.SemaphoreType.DMA,
          pltpu.SemaphoreType.DMA,
          pltpu.SemaphoreType.DMA,
      ],
  )
  return run(sentence_batch.astype(jnp.int32).reshape(N * L), emb_table)


# speedup vs baseline: 2.4468x; 1.2682x over previous
import jax
import jax.numpy as jnp
from jax import lax
from jax.experimental import pallas as pl
from jax.experimental.pallas import tpu as pltpu
from jax.experimental.pallas import tpu_sc as plsc

N = 128
L = 200
PATCH = 16
IMG = 384
EMB_DIM = 768
ROWS = 24
LANES = 16
VECS = EMB_DIM // LANES

NUM_CORES = 2
NUM_SUBCORES = 16
NUM_WORKERS = NUM_CORES * NUM_SUBCORES
SPW = N // NUM_WORKERS


def _tanh16(x):
  return 1.0 - 2.0 / (jnp.exp(2.0 * x) + 1.0)


def _sc_body(tok_hbm, table_hbm, out_hbm,
             tok_v, rows0, rows1, band_a0, band_a1, band_b,
             gs0, gs1, ws0, ws1, rsem):
  wid = lax.axis_index("s") * NUM_CORES + lax.axis_index("c")
  rows_bufs = (rows0, rows1)
  gsems = (gs0, gs1)
  band_bufs = (band_a0, band_a1)
  wsems = (ws0, ws1)

  tok_off = pl.multiple_of(wid * (SPW * L), 8)
  pltpu.sync_copy(tok_hbm.at[pl.ds(tok_off, SPW * L)], tok_v)

  def idx(i, start, count):
    return tok_v.at[pl.ds(pl.multiple_of(i * L + start, 8), count)]

  def gather(i, r, buf, sem):
    count = 32 if r == 7 else ROWS
    return pltpu.make_async_copy(
        table_hbm.at[idx(i, r * ROWS, count)], buf.at[pl.ds(0, count)], sem)

  def band_write(n, r, buf, sem):
    return pltpu.make_async_copy(
        buf, out_hbm.at[n, :, pl.ds(pl.multiple_of(r * PATCH, 16), PATCH), :],
        sem)

  def build_cols(rows, band, lo, hi, off=0):
    def do_col(c, _):
      cbase = pl.multiple_of(c * PATCH, 16)

      @plsc.parallel_loop(0, VECS, unroll=8)
      def _(j):
        ch = j // PATCH
        y = j - ch * PATCH
        band[ch, y, pl.ds(cbase, LANES)] = _tanh16(
            rows[c + off, pl.ds(pl.multiple_of(j * LANES, 16), LANES)])

      return 0
    lax.fori_loop(lo, hi, do_col, 0)

  def do_sample(i, _):
    n = wid * SPW + i

    for r in range(8):
      if r < 7:
        gather(i, r + 1, rows_bufs[(r + 1) % 2], gsems[(r + 1) % 2]).start()
      gather(i, r, rows_bufs[r % 2], gsems[r % 2]).wait()

      if r >= 2:
        band_write(n, r, band_bufs[r % 2], wsems[r % 2]).wait()
      else:
        @pl.when(i > 0)
        def _():
          band_write(n, r, band_bufs[r % 2], wsems[r % 2]).wait()

      build_cols(rows_bufs[r % 2], band_bufs[r % 2], 0, ROWS)
      band_write(n, r, band_bufs[r % 2], wsems[r % 2]).start()

    @pl.when(i + 1 < SPW)
    def _():
      gather(i + 1, 0, rows0, gs0).start()

    @pl.when(i > 0)
    def _():
      for _k in range(ROWS - 9):
        band_write(n, 9, band_b, rsem).wait()

    @plsc.parallel_loop(0, VECS, unroll=8)
    def _(j):
      ch = j // PATCH
      y = j - ch * PATCH
      band_b[ch, y, pl.ds(0, LANES)] = _tanh16(
          rows1[31, pl.ds(pl.multiple_of(j * LANES, 16), LANES)])

    def fan_col(c, _):
      cbase = pl.multiple_of(c * PATCH, 16)

      @plsc.parallel_loop(0, VECS, unroll=8)
      def _(j):
        ch = j // PATCH
        y = j - ch * PATCH
        band_b[ch, y, pl.ds(cbase, LANES)] = band_b[ch, y, pl.ds(0, LANES)]

      return 0
    lax.fori_loop(1, ROWS, fan_col, 0)

    def write_rep(r, _):
      band_write(n, r, band_b, rsem).start()
      return 0
    lax.fori_loop(9, ROWS, write_rep, 0)

    band_write(n, 8, band_a0, ws0).wait()
    build_cols(rows1, band_a0, 0, 7, off=ROWS)

    def fill_199(c, _):
      cbase = pl.multiple_of(c * PATCH, 16)

      @plsc.parallel_loop(0, VECS, unroll=8)
      def _(j):
        ch = j // PATCH
        y = j - ch * PATCH
        band_a0[ch, y, pl.ds(cbase, LANES)] = band_b[ch, y, pl.ds(0, LANES)]

      return 0
    lax.fori_loop(7, ROWS, fill_199, 0)
    band_write(n, 8, band_a0, ws0).start()
    return 0

  gather(0, 0, rows0, gs0).start()
  lax.fori_loop(0, SPW, do_sample, 0)

  last = wid * SPW + (SPW - 1)
  band_write(last, 8, band_a0, ws0).wait()
  band_write(last, 7, band_a1, ws1).wait()
  for _k in range(ROWS - 9):
    band_write(last, 9, band_b, rsem).wait()


@jax.jit
def kernel(sentence_batch, emb_table):
  mesh = plsc.VectorSubcoreMesh(core_axis_name="c", subcore_axis_name="s",
                                num_cores=NUM_CORES,
                                num_subcores=NUM_SUBCORES)
  run = pl.kernel(
      _sc_body,
      out_type=jax.ShapeDtypeStruct((N, 3, IMG, IMG), jnp.float32),
      mesh=mesh,
      scratch_types=[
          pltpu.VMEM((SPW * L,), jnp.int32),
          pltpu.VMEM((ROWS, EMB_DIM), jnp.float32),
          pltpu.VMEM((ROWS + 8, EMB_DIM), jnp.float32),
          pltpu.VMEM((3, PATCH, IMG), jnp.float32),
          pltpu.VMEM((3, PATCH, IMG), jnp.float32),
          pltpu.VMEM((3, PATCH, IMG), jnp.float32),
          pltpu.SemaphoreType.DMA,
          pltpu.SemaphoreType.DMA,
          pltpu.SemaphoreType.DMA,
          pltpu.SemaphoreType.DMA,
          pltpu.SemaphoreType.DMA,
      ],
  )
  return run(sentence_batch.astype(jnp.int32).reshape(N * L), emb_table)
